# baseline (speedup 1.0000x reference)
import functools
import jax
import jax.numpy as jnp
from jax import lax
from jax.experimental import pallas as pl
from jax.experimental.pallas import tpu as pltpu
from jax.experimental.pallas import tpu_sc as plsc

_H = 32
_W = 32
_F = 128
_BS = 16


def _sc_body(col_hbm, row_hbm, out_hbm, colv, rowv, buf, sem):
    cid = lax.axis_index("c")
    sid = lax.axis_index("s")
    wid = sid * 2 + cid

    pltpu.sync_copy(col_hbm, colv)
    pltpu.sync_copy(row_hbm, rowv)

    for i in range(_W):
        for k in range(_F // 16):
            buf[i, pl.ds(k * 16, 16)] = colv[i, pl.ds(k * 16, 16)]
    for k in range(_F // 16):
        v = rowv[wid, pl.ds(k * 16, 16)]
        for i in range(_W):
            buf[i, pl.ds(_F + k * 16, 16)] = v

    copies = [
        pltpu.async_copy(buf, out_hbm.at[b, wid], sem) for b in range(_BS)
    ]
    for cp in copies:
        cp.wait()


def _sc_call(colw, roww):
    mesh = plsc.VectorSubcoreMesh(core_axis_name="c", subcore_axis_name="s")
    kfn = functools.partial(
        pl.kernel,
        mesh=mesh,
        out_type=jax.ShapeDtypeStruct((_BS, _H, _W, 2 * _F), jnp.float32),
        scratch_types=[
            pltpu.VMEM((_W, _F), jnp.float32),
            pltpu.VMEM((_H, _F), jnp.float32),
            pltpu.VMEM((_W, 2 * _F), jnp.float32),
            pltpu.SemaphoreType.DMA,
        ],
        compiler_params=pltpu.CompilerParams(use_tc_tiling_on_sc=True),
    )(_sc_body)
    return kfn(colw, roww)


def kernel(mask, row_weight, col_weight):
    bs, h, w = mask.shape
    out = _sc_call(col_weight[:w], row_weight[:h])
    return jnp.transpose(out, (0, 3, 1, 2))
